# baseline (speedup 1.0000x reference)
import jax
import jax.numpy as jnp
from jax import lax
from jax.experimental import pallas as pl
from jax.experimental.pallas import tpu as pltpu

VOCAB = 100000
EMBED = 32
HIDDEN = 64
NTOK = 256
LANES = 128

_VTILE = 12544


def _mlp_body(ids_ref, oh_ref, embed_t_ref, w1_t_ref, b1_ref, wh_t_ref,
              out_ref, xch_ref, h_ref, sem):
    @pl.when(pl.program_id(0) == 0)
    def _():
        def issue(i, carry):
            r = ids_ref[i]
            c0 = pl.multiple_of((r // LANES) * LANES, LANES)
            pltpu.make_async_copy(
                embed_t_ref.at[:, pl.ds(c0, LANES)], xch_ref.at[i], sem
            ).start()
            return carry

        lax.fori_loop(0, NTOK, issue, 0, unroll=16)

        def drain(i, carry):
            pltpu.make_async_copy(
                embed_t_ref.at[:, pl.ds(0, LANES)], xch_ref.at[i], sem
            ).wait()
            return carry

        lax.fori_loop(0, NTOK, drain, 0, unroll=16)

        x = jnp.sum(xch_ref[...] * oh_ref[...][:, None, :], axis=-1)
        h_ref[...] = lax.dot_general(
            x, w1_t_ref[...],
            (((1,), (0,)), ((), ())),
            preferred_element_type=jnp.float32,
        ) + b1_ref[...]

    out_ref[...] = lax.dot_general(
        h_ref[...], wh_t_ref[...],
        (((1,), (0,)), ((), ())),
        preferred_element_type=jnp.float32,
    )


def _mlp_tc(ids, oh, embed_t, w1_t, b1_2d, wh_t, interpret=False):
    grid = (pl.cdiv(VOCAB, _VTILE),)
    return pl.pallas_call(
        _mlp_body,
        grid=grid,
        in_specs=[
            pl.BlockSpec(memory_space=pltpu.MemorySpace.SMEM),
            pl.BlockSpec((NTOK, LANES), lambda i: (0, 0)),
            pl.BlockSpec(memory_space=pltpu.MemorySpace.HBM),
            pl.BlockSpec((EMBED, HIDDEN), lambda i: (0, 0)),
            pl.BlockSpec((1, HIDDEN), lambda i: (0, 0)),
            pl.BlockSpec((HIDDEN, _VTILE), lambda i: (0, i)),
        ],
        out_specs=pl.BlockSpec((NTOK, _VTILE), lambda i: (0, i)),
        out_shape=jax.ShapeDtypeStruct((NTOK, VOCAB), jnp.float32),
        scratch_shapes=[
            pltpu.VMEM((NTOK, EMBED, LANES), jnp.float32),
            pltpu.VMEM((NTOK, HIDDEN), jnp.float32),
            pltpu.SemaphoreType.DMA,
        ],
        interpret=interpret,
    )(ids, oh, embed_t, w1_t, b1_2d, wh_t)


def kernel(input_ids, embed, W1, b1, Wh):
    B, S = input_ids.shape
    ids = input_ids.reshape(NTOK).astype(jnp.int32)
    oh = (ids[:, None] % LANES == jnp.arange(LANES)[None, :]).astype(jnp.float32)
    logits = _mlp_tc(ids, oh, embed.T, W1.T, b1.reshape(1, HIDDEN), Wh.T)
    return logits.reshape(B, S, VOCAB)


# speedup vs baseline: 1.0016x; 1.0016x over previous
import jax
import jax.numpy as jnp
from jax import lax
from jax.experimental import pallas as pl
from jax.experimental.pallas import tpu as pltpu

VOCAB = 100000
EMBED = 32
HIDDEN = 64
NTOK = 256
LANES = 128

_VTILE = 12288


def _mlp_body(ids_ref, oh_ref, embed_t_ref, w1_t_ref, b1_ref, wh_t_ref,
              out_ref, xch_ref, h_ref, sem):
    @pl.when(pl.program_id(0) == 0)
    def _():
        def issue(i, carry):
            r = ids_ref[i]
            c0 = pl.multiple_of((r // LANES) * LANES, LANES)
            pltpu.make_async_copy(
                embed_t_ref.at[:, pl.ds(c0, LANES)], xch_ref.at[i], sem
            ).start()
            return carry

        lax.fori_loop(0, NTOK, issue, 0, unroll=16)

        def drain(i, carry):
            pltpu.make_async_copy(
                embed_t_ref.at[:, pl.ds(0, LANES)], xch_ref.at[i], sem
            ).wait()
            return carry

        lax.fori_loop(0, NTOK, drain, 0, unroll=16)

        sel = oh_ref[...][:, None, :] > 0.5
        x = jnp.sum(jnp.where(sel, xch_ref[...], 0.0), axis=-1)
        h_ref[...] = lax.dot_general(
            x, w1_t_ref[...],
            (((1,), (0,)), ((), ())),
            preferred_element_type=jnp.float32,
        ) + b1_ref[...]

    out_ref[...] = lax.dot_general(
        h_ref[...], wh_t_ref[...],
        (((1,), (0,)), ((), ())),
        preferred_element_type=jnp.float32,
    )


def _mlp_tc(ids, oh, embed_t, w1_t, b1_2d, wh_t, interpret=False):
    grid = (pl.cdiv(VOCAB, _VTILE),)
    return pl.pallas_call(
        _mlp_body,
        grid=grid,
        in_specs=[
            pl.BlockSpec(memory_space=pltpu.MemorySpace.SMEM),
            pl.BlockSpec((NTOK, LANES), lambda i: (0, 0)),
            pl.BlockSpec(memory_space=pltpu.MemorySpace.HBM),
            pl.BlockSpec((EMBED, HIDDEN), lambda i: (0, 0)),
            pl.BlockSpec((1, HIDDEN), lambda i: (0, 0)),
            pl.BlockSpec((HIDDEN, _VTILE), lambda i: (0, i)),
        ],
        out_specs=pl.BlockSpec((NTOK, _VTILE), lambda i: (0, i)),
        out_shape=jax.ShapeDtypeStruct((NTOK, VOCAB), jnp.float32),
        scratch_shapes=[
            pltpu.VMEM((NTOK, EMBED, LANES), jnp.float32),
            pltpu.VMEM((NTOK, HIDDEN), jnp.float32),
            pltpu.SemaphoreType.DMA,
        ],
        interpret=interpret,
    )(ids, oh, embed_t, w1_t, b1_2d, wh_t)


def kernel(input_ids, embed, W1, b1, Wh):
    B, S = input_ids.shape
    ids = input_ids.reshape(NTOK).astype(jnp.int32)
    oh = (ids[:, None] % LANES == jnp.arange(LANES)[None, :]).astype(jnp.float32)
    logits = _mlp_tc(ids, oh, embed.T, W1.T, b1.reshape(1, HIDDEN), Wh.T)
    return logits.reshape(B, S, VOCAB)


# speedup vs baseline: 1.0057x; 1.0040x over previous
import jax
import jax.numpy as jnp
from jax import lax
from jax.experimental import pallas as pl
from jax.experimental.pallas import tpu as pltpu

VOCAB = 100000
EMBED = 32
HIDDEN = 64
NTOK = 256
LANES = 128

_VTILE = 12288


def _mlp_body(ids_ref, oh_ref, embed_t_ref, w1_t_ref, b1_ref, wh_t_ref,
              out_ref, xch_ref, h_ref, sem):
    @pl.when(pl.program_id(0) == 0)
    def _():
        def issue(i, carry):
            r = ids_ref[i]
            c0 = pl.multiple_of((r // LANES) * LANES, LANES)
            pltpu.make_async_copy(
                embed_t_ref.at[:, pl.ds(c0, LANES)], xch_ref.at[i], sem
            ).start()
            return carry

        lax.fori_loop(0, NTOK, issue, 0, unroll=16)

        def drain(i, carry):
            pltpu.make_async_copy(
                embed_t_ref.at[:, pl.ds(0, LANES)], xch_ref.at[i], sem
            ).wait()
            return carry

        lax.fori_loop(0, NTOK, drain, 0, unroll=16)

        sel = oh_ref[...][:, None, :] > 0.5
        x = jnp.sum(jnp.where(sel, xch_ref[...], 0.0), axis=-1)
        h_ref[...] = lax.dot_general(
            x, w1_t_ref[...],
            (((1,), (0,)), ((), ())),
            preferred_element_type=jnp.float32,
        ) + b1_ref[...]

    out_ref[...] = lax.dot_general(
        h_ref[...], wh_t_ref[...],
        (((1,), (0,)), ((), ())),
        preferred_element_type=jnp.float32,
    )


def _mlp_tc(ids, oh, embed_t, w1_t, b1_2d, wh_t):
    grid = (pl.cdiv(VOCAB, _VTILE),)
    return pl.pallas_call(
        _mlp_body,
        grid=grid,
        in_specs=[
            pl.BlockSpec(memory_space=pltpu.MemorySpace.SMEM),
            pl.BlockSpec((NTOK, LANES), lambda i: (0, 0)),
            pl.BlockSpec(memory_space=pltpu.MemorySpace.HBM),
            pl.BlockSpec((EMBED, HIDDEN), lambda i: (0, 0)),
            pl.BlockSpec((1, HIDDEN), lambda i: (0, 0)),
            pl.BlockSpec((HIDDEN, _VTILE), lambda i: (0, i)),
        ],
        out_specs=pl.BlockSpec((NTOK, _VTILE), lambda i: (0, i)),
        out_shape=jax.ShapeDtypeStruct((NTOK, VOCAB), jnp.float32),
        scratch_shapes=[
            pltpu.VMEM((NTOK, EMBED, LANES), jnp.float32),
            pltpu.VMEM((NTOK, HIDDEN), jnp.float32),
            pltpu.SemaphoreType.DMA,
        ],
    )(ids, oh, embed_t, w1_t, b1_2d, wh_t)


def kernel(input_ids, embed, W1, b1, Wh):
    B, S = input_ids.shape
    ids = input_ids.reshape(NTOK).astype(jnp.int32)
    oh = (ids[:, None] % LANES == jnp.arange(LANES)[None, :]).astype(jnp.float32)
    logits = _mlp_tc(ids, oh, embed.T, W1.T, b1.reshape(1, HIDDEN), Wh.T)
    return logits.reshape(B, S, VOCAB)
